# baseline (speedup 1.0000x reference)
import jax
import jax.numpy as jnp
from jax.experimental import pallas as pl
from jax.experimental.pallas import tpu as pltpu


def _encode_hidden_kernel(q_ref, wq_ref, bq_ref, w1_ref, b1_ref, ce_ref,
                          h_ref, acc_ref):
    k = pl.program_id(0)

    @pl.when(k == 0)
    def _init():
        acc_ref[...] = jnp.zeros_like(acc_ref)

    batch = q_ref.shape[0]
    embed = wq_ref.shape[0]
    bk = q_ref.shape[-1]
    q = q_ref[...].reshape(batch, bk).astype(jnp.bfloat16)
    wq = wq_ref[...].reshape(embed, bk).astype(jnp.bfloat16)
    acc_ref[...] += jax.lax.dot_general(
        q, wq, (((1,), (1,)), ((), ())), preferred_element_type=jnp.float32)

    @pl.when(k == pl.num_programs(0) - 1)
    def _finish():
        embed = bq_ref.shape[-1]
        qe = (acc_ref[...] + bq_ref[...]).astype(jnp.bfloat16)
        ce = ce_ref[...].astype(jnp.bfloat16)
        w1 = w1_ref[...]
        w1a = w1[:, :embed].astype(jnp.bfloat16)
        w1b = w1[:, embed:].astype(jnp.bfloat16)
        pre = jax.lax.dot_general(
            qe, w1a, (((1,), (1,)), ((), ())),
            preferred_element_type=jnp.float32)
        pre += jax.lax.dot_general(
            ce, w1b, (((1,), (1,)), ((), ())),
            preferred_element_type=jnp.float32)
        pre += b1_ref[...]
        h_ref[...] = jnp.maximum(pre, 0.0)


def _output_kernel(h_ref, w2_ref, b2_ref, out_ref):
    batch = h_ref.shape[0]
    bn = w2_ref.shape[0]
    h = h_ref[...].astype(jnp.bfloat16)
    w2 = w2_ref[...].astype(jnp.bfloat16)
    res = jax.lax.dot_general(
        h, w2, (((1,), (1,)), ((), ())),
        preferred_element_type=jnp.float32) + b2_ref[...].reshape(1, bn)
    out_ref[...] = res.reshape(batch, 1, 1, bn)


def kernel(query, W_q, b_q, W1, b1, W2, b2, top_k):
    del top_k
    batch, vocab = query.shape
    embed = W_q.shape[0]
    hidden = W1.shape[0]

    ce = jax.random.normal(jax.random.key(42), (batch, embed),
                           dtype=jnp.float32)

    bk = 2500
    bn = 2000
    nk = vocab // bk
    nn = vocab // bn

    q4 = query.reshape(batch, nk, 1, bk)
    wq4 = W_q.reshape(embed, nk, 1, bk)

    h = pl.pallas_call(
        _encode_hidden_kernel,
        grid=(nk,),
        in_specs=[
            pl.BlockSpec((batch, 1, 1, bk), lambda k: (0, k, 0, 0)),
            pl.BlockSpec((embed, 1, 1, bk), lambda k: (0, k, 0, 0)),
            pl.BlockSpec((1, embed), lambda k: (0, 0)),
            pl.BlockSpec((hidden, 2 * embed), lambda k: (0, 0)),
            pl.BlockSpec((1, hidden), lambda k: (0, 0)),
            pl.BlockSpec((batch, embed), lambda k: (0, 0)),
        ],
        out_specs=pl.BlockSpec((batch, hidden), lambda k: (0, 0)),
        out_shape=jax.ShapeDtypeStruct((batch, hidden), jnp.float32),
        scratch_shapes=[pltpu.VMEM((batch, embed), jnp.float32)],
        compiler_params=pltpu.CompilerParams(
            dimension_semantics=("arbitrary",)),
    )(q4, wq4, b_q.reshape(1, embed), W1, b1.reshape(1, hidden), ce)

    out4 = pl.pallas_call(
        _output_kernel,
        grid=(nn,),
        in_specs=[
            pl.BlockSpec((batch, hidden), lambda n: (0, 0)),
            pl.BlockSpec((bn, hidden), lambda n: (n, 0)),
            pl.BlockSpec((1, 1, bn), lambda n: (n, 0, 0)),
        ],
        out_specs=pl.BlockSpec((batch, 1, 1, bn), lambda n: (0, n, 0, 0)),
        out_shape=jax.ShapeDtypeStruct((batch, nn, 1, bn), jnp.float32),
        compiler_params=pltpu.CompilerParams(
            dimension_semantics=("arbitrary",)),
    )(h, W2, b2.reshape(nn, 1, bn))

    return out4.reshape(batch, vocab)


# speedup vs baseline: 2.9882x; 2.9882x over previous
import functools

import jax
import jax.numpy as jnp
from jax.experimental import pallas as pl
from jax.experimental.pallas import tpu as pltpu

_BK = 4096
_BN = 4096


def _encode_hidden_kernel(vocab, q_ref, wq_ref, bq_ref, w1_ref, b1_ref,
                          ce_ref, h_ref, acc_ref):
    k = pl.program_id(0)
    nk = pl.num_programs(0)
    bk = q_ref.shape[-1]
    tail = vocab - (nk - 1) * bk

    @pl.when(k == 0)
    def _init():
        acc_ref[...] = jnp.zeros_like(acc_ref)

    @pl.when(k < nk - 1)
    def _full_step():
        q = q_ref[...].astype(jnp.bfloat16)
        wq = wq_ref[...].astype(jnp.bfloat16)
        acc_ref[...] += jax.lax.dot_general(
            q, wq, (((1,), (1,)), ((), ())),
            preferred_element_type=jnp.float32)

    @pl.when(k == nk - 1)
    def _last_step():
        q = q_ref[:, :tail].astype(jnp.bfloat16)
        wq = wq_ref[:, :tail].astype(jnp.bfloat16)
        acc = acc_ref[...] + jax.lax.dot_general(
            q, wq, (((1,), (1,)), ((), ())),
            preferred_element_type=jnp.float32)
        embed = bq_ref.shape[-1]
        qe = (acc + bq_ref[...]).astype(jnp.bfloat16)
        ce = ce_ref[...].astype(jnp.bfloat16)
        w1 = w1_ref[...]
        w1a = w1[:, :embed].astype(jnp.bfloat16)
        w1b = w1[:, embed:].astype(jnp.bfloat16)
        pre = jax.lax.dot_general(
            qe, w1a, (((1,), (1,)), ((), ())),
            preferred_element_type=jnp.float32)
        pre += jax.lax.dot_general(
            ce, w1b, (((1,), (1,)), ((), ())),
            preferred_element_type=jnp.float32)
        pre += b1_ref[...]
        h_ref[...] = jnp.maximum(pre, 0.0)


def _output_kernel(h_ref, w2_ref, b2_ref, out_ref):
    h = h_ref[...].astype(jnp.bfloat16)
    w2 = w2_ref[...].astype(jnp.bfloat16)
    out_ref[...] = jax.lax.dot_general(
        h, w2, (((1,), (1,)), ((), ())),
        preferred_element_type=jnp.float32) + b2_ref[...]


def kernel(query, W_q, b_q, W1, b1, W2, b2, top_k):
    del top_k
    batch, vocab = query.shape
    embed = W_q.shape[0]
    hidden = W1.shape[0]

    ce = jax.random.normal(jax.random.key(42), (batch, embed),
                           dtype=jnp.float32)

    nk = pl.cdiv(vocab, _BK)
    h = pl.pallas_call(
        functools.partial(_encode_hidden_kernel, vocab),
        grid=(nk,),
        in_specs=[
            pl.BlockSpec((batch, _BK), lambda k: (0, k)),
            pl.BlockSpec((embed, _BK), lambda k: (0, k)),
            pl.BlockSpec((1, embed), lambda k: (0, 0)),
            pl.BlockSpec((hidden, 2 * embed), lambda k: (0, 0)),
            pl.BlockSpec((1, hidden), lambda k: (0, 0)),
            pl.BlockSpec((batch, embed), lambda k: (0, 0)),
        ],
        out_specs=pl.BlockSpec((batch, hidden), lambda k: (0, 0)),
        out_shape=jax.ShapeDtypeStruct((batch, hidden), jnp.float32),
        scratch_shapes=[pltpu.VMEM((batch, embed), jnp.float32)],
        compiler_params=pltpu.CompilerParams(
            dimension_semantics=("arbitrary",)),
    )(query, W_q, b_q.reshape(1, embed), W1, b1.reshape(1, hidden), ce)

    nn = pl.cdiv(vocab, _BN)
    out = pl.pallas_call(
        _output_kernel,
        grid=(nn,),
        in_specs=[
            pl.BlockSpec((batch, hidden), lambda n: (0, 0)),
            pl.BlockSpec((_BN, hidden), lambda n: (n, 0)),
            pl.BlockSpec((1, _BN), lambda n: (0, n)),
        ],
        out_specs=pl.BlockSpec((batch, _BN), lambda n: (0, n)),
        out_shape=jax.ShapeDtypeStruct((batch, vocab), jnp.float32),
        compiler_params=pltpu.CompilerParams(
            dimension_semantics=("arbitrary",)),
    )(h, W2, b2.reshape(1, vocab))

    return out
